# baseline (speedup 1.0000x reference)
import functools

import jax
import jax.numpy as jnp
import numpy as np
from jax.experimental import pallas as pl
from jax.experimental.pallas import tpu as pltpu

DIM = 128
L_LITS = 20000

_BM = 640


def _pairswap(y, bm):
    down = pltpu.roll(y, bm - 1, 0)
    up = pltpu.roll(y, 1, 0)
    row = jax.lax.broadcasted_iota(jnp.int32, (bm, DIM), 0)
    return jnp.where((row % 2) == 0, down, up)


def _m1_body(x_ref, w1_ref, b1_ref, w2_ref, b2_ref, nw1_ref, nb1_ref,
             nw2_ref, nb2_ref, o_ref):
    x = x_ref[...]
    h = jnp.maximum(
        jnp.dot(x, w1_ref[...], preferred_element_type=jnp.float32)
        + b1_ref[...], 0.0)
    y = (jnp.dot(h, w2_ref[...], preferred_element_type=jnp.float32)
         + b2_ref[...])
    inv = _pairswap(y, x.shape[0])
    z = jnp.concatenate([y, inv], axis=1)
    h2 = jnp.maximum(
        jnp.dot(z, nw1_ref[...], preferred_element_type=jnp.float32)
        + nb1_ref[...], 0.0)
    o_ref[...] = (jnp.dot(h2, nw2_ref[...], preferred_element_type=jnp.float32)
                  + nb2_ref[...])


def _mlp_fused_l2c(x, w1, b1, w2, b2, nw1, nb1, nw2, nb2):
    n = x.shape[0]
    grid = n // _BM
    full = lambda s: pl.BlockSpec(s, lambda i: (0, 0))
    return pl.pallas_call(
        _m1_body,
        grid=(grid,),
        in_specs=[
            pl.BlockSpec((_BM, DIM), lambda i: (i, 0)),
            full((DIM, DIM)), full((1, DIM)), full((DIM, DIM)), full((1, DIM)),
            full((2 * DIM, DIM)), full((1, DIM)), full((DIM, DIM)), full((1, DIM)),
        ],
        out_specs=pl.BlockSpec((_BM, DIM), lambda i: (i, 0)),
        out_shape=jax.ShapeDtypeStruct((n, DIM), jnp.float32),
    )(x, w1, b1.reshape(1, DIM), w2, b2.reshape(1, DIM),
      nw1, nb1.reshape(1, DIM), nw2, nb2.reshape(1, DIM))


def _m2_body(s_ref, m_ref, w1_ref, b1_ref, w2_ref, b2_ref, o_ref):
    s = s_ref[...]
    m = m_ref[...]
    x = jnp.log(s + 1e-12) + jnp.where(s > 0, m, 0.0)
    h = jnp.maximum(
        jnp.dot(x, w1_ref[...], preferred_element_type=jnp.float32)
        + b1_ref[...], 0.0)
    o_ref[...] = (jnp.dot(h, w2_ref[...], preferred_element_type=jnp.float32)
                  + b2_ref[...])


def _mlp_fused_c2l(s, m, w1, b1, w2, b2):
    n = s.shape[0]
    grid = n // _BM
    full = lambda sh: pl.BlockSpec(sh, lambda i: (0, 0))
    return pl.pallas_call(
        _m2_body,
        grid=(grid,),
        in_specs=[
            pl.BlockSpec((_BM, DIM), lambda i: (i, 0)),
            pl.BlockSpec((_BM, DIM), lambda i: (i, 0)),
            full((DIM, DIM)), full((1, DIM)), full((DIM, DIM)), full((1, DIM)),
        ],
        out_specs=pl.BlockSpec((_BM, DIM), lambda i: (i, 0)),
        out_shape=jax.ShapeDtypeStruct((n, DIM), jnp.float32),
    )(s, m, w1, b1.reshape(1, DIM), w2, b2.reshape(1, DIM))


def _m3_body(x_ref, w1_ref, b1_ref, w2_ref, b2_ref, o_ref):
    x = x_ref[...]
    h = jnp.maximum(
        jnp.dot(x, w1_ref[...], preferred_element_type=jnp.float32)
        + b1_ref[...], 0.0)
    y = (jnp.dot(h, w2_ref[...], preferred_element_type=jnp.float32)
         + b2_ref[...])
    d = y - _pairswap(y, x.shape[0])
    o_ref[...] = jax.nn.sigmoid(d)


def _readout(x, w1, b1, w2, b2):
    w2p = jnp.pad(w2, ((0, 0), (0, DIM - 1)))
    b2p = jnp.pad(b2.reshape(1, 1), ((0, 0), (0, DIM - 1)))
    n = x.shape[0]
    bm = 400
    grid = n // bm
    full = lambda sh: pl.BlockSpec(sh, lambda i: (0, 0))
    out = pl.pallas_call(
        _m3_body,
        grid=(grid,),
        in_specs=[
            pl.BlockSpec((bm, DIM), lambda i: (i, 0)),
            full((DIM, DIM)), full((1, DIM)), full((DIM, DIM)), full((1, DIM)),
        ],
        out_specs=pl.BlockSpec((bm, DIM), lambda i: (i, 0)),
        out_shape=jax.ShapeDtypeStruct((n, DIM), jnp.float32),
    )(x, w1, b1.reshape(1, DIM), w2p, b2p)
    return out[:, 0].reshape(-1, 2)


def _segsum(feat, g, sd, n_out):
    return jax.ops.segment_sum(feat[g], sd, num_segments=n_out,
                               indices_are_sorted=True)


def _seg_logsumexp_parts(src_sorted, sd, n_out):
    seg_max = jax.ops.segment_max(src_sorted, sd, num_segments=n_out,
                                  indices_are_sorted=True)
    seg_max = jnp.where(jnp.isfinite(seg_max), seg_max, 0.0)
    seg_sum = jax.ops.segment_sum(jnp.exp(src_sorted - seg_max[sd]), sd,
                                  num_segments=n_out, indices_are_sorted=True)
    return seg_sum, seg_max


def kernel(sign_l_edge_index, c2l_msg_repeat_index, c2l_msg_scatter_index,
           l2c_msg_aggr_repeat_index, l2c_msg_aggr_scatter_index,
           l2c_msg_scatter_index, num_edges, l_size,
           c2l_init, l2c_init,
           c2l_W1, c2l_b1, c2l_W2, c2l_b2,
           l2c_W1, l2c_b1, l2c_W2, l2c_b2,
           nm_W1, nm_b1, nm_W2, nm_b2,
           ro_W1, ro_b1, ro_W2, ro_b2):
    E = sign_l_edge_index.shape[0]
    denom = np.sqrt(DIM)

    perm1 = jnp.argsort(c2l_msg_scatter_index)
    sd1 = c2l_msg_scatter_index[perm1]
    g1 = c2l_msg_repeat_index[perm1]

    perm2 = jnp.argsort(l2c_msg_aggr_scatter_index)
    sd2 = l2c_msg_aggr_scatter_index[perm2]
    g2 = l2c_msg_aggr_repeat_index[perm2]

    perm3 = jnp.argsort(l2c_msg_scatter_index)
    sd3 = l2c_msg_scatter_index[perm3]

    perm4 = jnp.argsort(sign_l_edge_index)
    sd4 = sign_l_edge_index[perm4]

    c2l_feat = jnp.tile(c2l_init / denom, (E, 1))

    for _ in range(2):
        c2l_msg = _segsum(c2l_feat, g1, sd1, E)
        l2c_feat = _mlp_fused_l2c(c2l_msg, l2c_W1, l2c_b1, l2c_W2, l2c_b2,
                                  nm_W1, nm_b1, nm_W2, nm_b2)
        l2c_aggr = _segsum(l2c_feat, g2, sd2, E)
        s, m = _seg_logsumexp_parts(l2c_aggr[perm3], sd3, E)
        c2l_feat = _mlp_fused_c2l(s, m, c2l_W1, c2l_b1, c2l_W2, c2l_b2)

    l_logit = _segsum(c2l_feat, perm4, sd4, L_LITS)
    out = _readout(l_logit, ro_W1, ro_b1, ro_W2, ro_b2)
    return out + 0.0 * (num_edges + l_size)


# speedup vs baseline: 1.0298x; 1.0298x over previous
import functools

import jax
import jax.numpy as jnp
import numpy as np
from jax import lax
from jax.experimental import pallas as pl
from jax.experimental.pallas import tpu as pltpu
from jax.experimental.pallas import tpu_sc as plsc

DIM = 128
L_LITS = 20000

_EB = 128
_NW = 32


def _make_segop(mode, n_out, r_tile, n_pass, e_pad):
    rw = n_out // _NW
    assert rw == r_tile * n_pass and r_tile % 8 == 0
    nu = _NW * n_pass
    nu_pad = -(-(nu + 1) // 8) * 8
    mesh = plsc.VectorSubcoreMesh(core_axis_name="c", subcore_axis_name="s")
    if mode == "sum":
        out_type = jax.ShapeDtypeStruct((n_out, DIM), jnp.float32)
    else:
        out_type = (jax.ShapeDtypeStruct((n_out, DIM), jnp.float32),
                    jax.ShapeDtypeStruct((n_out, DIM), jnp.float32))
    scratch = [
        pltpu.VMEM((r_tile, DIM), jnp.float32),
        pltpu.VMEM_SHARED((384, DIM), jnp.float32),
        pltpu.VMEM((_EB, DIM), jnp.float32),
        pltpu.VMEM((_EB,), jnp.int32),
        pltpu.VMEM((_EB, 16), jnp.int32),
        pltpu.VMEM((nu_pad, 16), jnp.int32),
        pltpu.SemaphoreType.DMA,
    ]
    if mode == "lse":
        scratch.insert(2, pltpu.VMEM((r_tile, DIM), jnp.float32))

    @functools.partial(pl.kernel, out_type=out_type, mesh=mesh,
                       compiler_params=pltpu.CompilerParams(
                           use_tc_tiling_on_sc=False),
                       scratch_types=scratch)
    def segop(feat, g, sd16, tb, *rest):
        if mode == "sum":
            (out, accum, zconst, rows_v, idx_v, sd_v, tb_v, sem) = rest
        else:
            (out_s, out_m, accum, zconst, amax, rows_v, idx_v, sd_v, tb_v,
             sem) = rest
        w = lax.axis_index("s") * 2 + lax.axis_index("c")
        pltpu.sync_copy(tb, tb_v)
        zv = jnp.zeros((16,), jnp.float32)

        def zbody(r, _):
            for j in range(DIM // 16):
                rows_v[r, pl.ds(j * 16, 16)] = zv
            return 0

        lax.fori_loop(0, _EB, zbody, 0)
        for off in (0, 128, 256):
            pltpu.sync_copy(rows_v, zconst.at[pl.ds(off, _EB)])

        def unit_bounds(p):
            u = w * n_pass + p
            lo = tb_v[u, pl.ds(0, 16)][0]
            hi = tb_v[u + 1, pl.ds(0, 16)][0]
            return lo, hi

        def gather_block(bs):
            pltpu.sync_copy(g.at[pl.ds(bs, _EB)], idx_v)
            pltpu.sync_copy(sd16.at[pl.ds(bs, _EB)], sd_v)
            pltpu.async_copy(feat.at[idx_v], rows_v, sem).wait()

        def edge_range(bs, lo, hi):
            i0 = jnp.maximum(lo - bs, 0)
            i1 = jnp.minimum(hi - bs, _EB)
            return i0, i1

        def p_loop_sum(p, _):
            lo, hi = unit_bounds(p)
            base_row = w * rw + p * r_tile
            pltpu.sync_copy(zconst.at[pl.ds(0, r_tile)], accum)
            lo_al = (lo // _EB) * _EB
            nblk = (hi - lo_al + _EB - 1) // _EB

            def blk(bi, _c):
                bs = lo_al + bi * _EB
                gather_block(bs)
                i0, i1 = edge_range(bs, lo, hi)

                def edge(i, _e):
                    ldi = sd_v[i, pl.ds(0, 16)][0] - base_row
                    for j in range(DIM // 16):
                        sl = pl.ds(j * 16, 16)
                        accum[ldi, sl] = accum[ldi, sl] + rows_v[i, sl]
                    return 0

                lax.fori_loop(i0, i1, edge, 0)
                return 0

            lax.fori_loop(0, nblk, blk, 0)
            pltpu.sync_copy(
                accum,
                out.at[pl.ds(pl.multiple_of(base_row, 8), r_tile)])
            return 0

        def p_loop_lse(p, _):
            lo, hi = unit_bounds(p)
            base_row = w * rw + p * r_tile
            pltpu.sync_copy(zconst.at[pl.ds(0, r_tile)], accum)
            lo_al = (lo // _EB) * _EB
            nblk = (hi - lo_al + _EB - 1) // _EB

            def blkA(bi, car):
                bs = lo_al + bi * _EB
                gather_block(bs)
                i0, i1 = edge_range(bs, lo, hi)

                def edge(i, car):
                    m, prev, started = car
                    sdi = sd_v[i, pl.ds(0, 16)][0]

                    @pl.when((started == 1) & (sdi != prev))
                    def _():
                        for j in range(DIM // 16):
                            amax[prev - base_row, pl.ds(j * 16, 16)] = m[j]

                    same = (started == 1) & (sdi == prev)
                    m2 = tuple(
                        jnp.where(same,
                                  jnp.maximum(m[j],
                                              rows_v[i, pl.ds(j * 16, 16)]),
                                  rows_v[i, pl.ds(j * 16, 16)])
                        for j in range(DIM // 16))
                    return (m2, sdi, jnp.int32(1))

                return lax.fori_loop(i0, i1, edge, car)

            car0 = (tuple(zv for _ in range(DIM // 16)), jnp.int32(-1),
                    jnp.int32(0))
            m, prev, started = lax.fori_loop(0, nblk, blkA, car0)

            @pl.when(started == 1)
            def _():
                for j in range(DIM // 16):
                    amax[prev - base_row, pl.ds(j * 16, 16)] = m[j]

            def blkB(bi, _c):
                bs = lo_al + bi * _EB
                gather_block(bs)
                i0, i1 = edge_range(bs, lo, hi)

                def edge(i, _e):
                    ldi = sd_v[i, pl.ds(0, 16)][0] - base_row
                    for j in range(DIM // 16):
                        sl = pl.ds(j * 16, 16)
                        accum[ldi, sl] = accum[ldi, sl] + jnp.exp(
                            rows_v[i, sl] - amax[ldi, sl])
                    return 0

                lax.fori_loop(i0, i1, edge, 0)
                return 0

            lax.fori_loop(0, nblk, blkB, 0)
            ob = pl.ds(pl.multiple_of(base_row, 8), r_tile)
            pltpu.sync_copy(accum, out_s.at[ob])
            pltpu.sync_copy(amax, out_m.at[ob])
            return 0

        lax.fori_loop(0, n_pass, p_loop_sum if mode == "sum" else p_loop_lse,
                      0)

    return segop


@functools.lru_cache(maxsize=None)
def _segop_fn(mode, n_out, r_tile, n_pass, e_pad):
    return jax.jit(_make_segop(mode, n_out, r_tile, n_pass, e_pad))


def _segop_sc(mode, feat, g, sd, n_out, r_tile, n_pass):
    e = g.shape[0]
    e_pad = e + _EB
    rw = n_out // _NW
    n_pass_i = n_pass
    nu = _NW * n_pass_i
    nu_pad = -(-(nu + 1) // 8) * 8
    u = jnp.arange(nu + 1, dtype=jnp.int32)
    row0 = (u // n_pass_i) * rw + (u % n_pass_i) * r_tile
    tb = jnp.searchsorted(sd, row0, side="left").astype(jnp.int32)
    tb = jnp.concatenate([tb, jnp.full((nu_pad - nu - 1,), e, jnp.int32)])
    tb = jnp.broadcast_to(tb[:, None], (nu_pad, 16))
    g_p = jnp.concatenate([g.astype(jnp.int32),
                           jnp.zeros((e_pad - e,), jnp.int32)])
    sd16 = jnp.broadcast_to(
        jnp.concatenate([sd.astype(jnp.int32),
                         jnp.zeros((e_pad - e,), jnp.int32)])[:, None],
        (e_pad, 16))
    fn = _segop_fn(mode, n_out, r_tile, n_pass_i, e_pad)
    return fn(feat, g_p, sd16, tb)


def _segsum_sc(feat, g, sd, n_out, r_tile, n_pass):
    return _segop_sc("sum", feat, g, sd, n_out, r_tile, n_pass)


def _seglse_sc(feat, g, sd, n_out, r_tile, n_pass):
    return _segop_sc("lse", feat, g, sd, n_out, r_tile, n_pass)


_BM = 640


def _pairswap(y, bm):
    down = pltpu.roll(y, bm - 1, 0)
    up = pltpu.roll(y, 1, 0)
    row = jax.lax.broadcasted_iota(jnp.int32, (bm, DIM), 0)
    return jnp.where((row % 2) == 0, down, up)


def _m1_body(x_ref, w1_ref, b1_ref, w2_ref, b2_ref, nw1_ref, nb1_ref,
             nw2_ref, nb2_ref, o_ref):
    x = x_ref[...]
    h = jnp.maximum(
        jnp.dot(x, w1_ref[...], preferred_element_type=jnp.float32)
        + b1_ref[...], 0.0)
    y = (jnp.dot(h, w2_ref[...], preferred_element_type=jnp.float32)
         + b2_ref[...])
    inv = _pairswap(y, x.shape[0])
    z = jnp.concatenate([y, inv], axis=1)
    h2 = jnp.maximum(
        jnp.dot(z, nw1_ref[...], preferred_element_type=jnp.float32)
        + nb1_ref[...], 0.0)
    o_ref[...] = (jnp.dot(h2, nw2_ref[...], preferred_element_type=jnp.float32)
                  + nb2_ref[...])


def _mlp_fused_l2c(x, w1, b1, w2, b2, nw1, nb1, nw2, nb2):
    n = x.shape[0]
    grid = n // _BM
    full = lambda s: pl.BlockSpec(s, lambda i: (0, 0))
    return pl.pallas_call(
        _m1_body,
        grid=(grid,),
        in_specs=[
            pl.BlockSpec((_BM, DIM), lambda i: (i, 0)),
            full((DIM, DIM)), full((1, DIM)), full((DIM, DIM)), full((1, DIM)),
            full((2 * DIM, DIM)), full((1, DIM)), full((DIM, DIM)), full((1, DIM)),
        ],
        out_specs=pl.BlockSpec((_BM, DIM), lambda i: (i, 0)),
        out_shape=jax.ShapeDtypeStruct((n, DIM), jnp.float32),
    )(x, w1, b1.reshape(1, DIM), w2, b2.reshape(1, DIM),
      nw1, nb1.reshape(1, DIM), nw2, nb2.reshape(1, DIM))


def _m2_body(s_ref, m_ref, w1_ref, b1_ref, w2_ref, b2_ref, o_ref):
    s = s_ref[...]
    m = m_ref[...]
    x = jnp.log(s + 1e-12) + jnp.where(s > 0, m, 0.0)
    h = jnp.maximum(
        jnp.dot(x, w1_ref[...], preferred_element_type=jnp.float32)
        + b1_ref[...], 0.0)
    o_ref[...] = (jnp.dot(h, w2_ref[...], preferred_element_type=jnp.float32)
                  + b2_ref[...])


def _mlp_fused_c2l(s, m, w1, b1, w2, b2):
    n = s.shape[0]
    grid = n // _BM
    full = lambda sh: pl.BlockSpec(sh, lambda i: (0, 0))
    return pl.pallas_call(
        _m2_body,
        grid=(grid,),
        in_specs=[
            pl.BlockSpec((_BM, DIM), lambda i: (i, 0)),
            pl.BlockSpec((_BM, DIM), lambda i: (i, 0)),
            full((DIM, DIM)), full((1, DIM)), full((DIM, DIM)), full((1, DIM)),
        ],
        out_specs=pl.BlockSpec((_BM, DIM), lambda i: (i, 0)),
        out_shape=jax.ShapeDtypeStruct((n, DIM), jnp.float32),
    )(s, m, w1, b1.reshape(1, DIM), w2, b2.reshape(1, DIM))


def _m3_body(x_ref, w1_ref, b1_ref, w2_ref, b2_ref, o_ref):
    x = x_ref[...]
    h = jnp.maximum(
        jnp.dot(x, w1_ref[...], preferred_element_type=jnp.float32)
        + b1_ref[...], 0.0)
    y = (jnp.dot(h, w2_ref[...], preferred_element_type=jnp.float32)
         + b2_ref[...])
    d = y - _pairswap(y, x.shape[0])
    o_ref[...] = jax.nn.sigmoid(d)


def _readout(x, w1, b1, w2, b2):
    w2p = jnp.pad(w2, ((0, 0), (0, DIM - 1)))
    b2p = jnp.pad(b2.reshape(1, 1), ((0, 0), (0, DIM - 1)))
    n = x.shape[0]
    bm = 400
    grid = n // bm
    full = lambda sh: pl.BlockSpec(sh, lambda i: (0, 0))
    out = pl.pallas_call(
        _m3_body,
        grid=(grid,),
        in_specs=[
            pl.BlockSpec((bm, DIM), lambda i: (i, 0)),
            full((DIM, DIM)), full((1, DIM)), full((DIM, DIM)), full((1, DIM)),
        ],
        out_specs=pl.BlockSpec((bm, DIM), lambda i: (i, 0)),
        out_shape=jax.ShapeDtypeStruct((n, DIM), jnp.float32),
    )(x, w1, b1.reshape(1, DIM), w2p, b2p)
    return out[:, 0].reshape(-1, 2)


def _segsum(feat, g, sd, n_out):
    return jax.ops.segment_sum(feat[g], sd, num_segments=n_out,
                               indices_are_sorted=True)


def _seg_logsumexp_parts(src_sorted, sd, n_out):
    seg_max = jax.ops.segment_max(src_sorted, sd, num_segments=n_out,
                                  indices_are_sorted=True)
    seg_max = jnp.where(jnp.isfinite(seg_max), seg_max, 0.0)
    seg_sum = jax.ops.segment_sum(jnp.exp(src_sorted - seg_max[sd]), sd,
                                  num_segments=n_out, indices_are_sorted=True)
    return seg_sum, seg_max


def kernel(sign_l_edge_index, c2l_msg_repeat_index, c2l_msg_scatter_index,
           l2c_msg_aggr_repeat_index, l2c_msg_aggr_scatter_index,
           l2c_msg_scatter_index, num_edges, l_size,
           c2l_init, l2c_init,
           c2l_W1, c2l_b1, c2l_W2, c2l_b2,
           l2c_W1, l2c_b1, l2c_W2, l2c_b2,
           nm_W1, nm_b1, nm_W2, nm_b2,
           ro_W1, ro_b1, ro_W2, ro_b2):
    E = sign_l_edge_index.shape[0]
    denom = np.sqrt(DIM)

    perm1 = jnp.argsort(c2l_msg_scatter_index)
    sd1 = c2l_msg_scatter_index[perm1]
    g1 = c2l_msg_repeat_index[perm1]

    perm2 = jnp.argsort(l2c_msg_aggr_scatter_index)
    sd2 = l2c_msg_aggr_scatter_index[perm2]
    g2 = l2c_msg_aggr_repeat_index[perm2]

    perm3 = jnp.argsort(l2c_msg_scatter_index)
    sd3 = l2c_msg_scatter_index[perm3]

    perm4 = jnp.argsort(sign_l_edge_index)
    sd4 = sign_l_edge_index[perm4]

    c2l_feat = jnp.tile(c2l_init / denom, (E, 1))

    for _ in range(2):
        c2l_msg = _segsum_sc(c2l_feat, g1, sd1, E, 200, 25)
        l2c_feat = _mlp_fused_l2c(c2l_msg, l2c_W1, l2c_b1, l2c_W2, l2c_b2,
                                  nm_W1, nm_b1, nm_W2, nm_b2)
        l2c_aggr = _segsum_sc(l2c_feat, g2, sd2, E, 200, 25)
        s, m = _seg_logsumexp_parts(l2c_aggr[perm3], sd3, E)
        c2l_feat = _mlp_fused_c2l(s, m, c2l_W1, c2l_b1, c2l_W2, c2l_b2)

    l_logit = _segsum_sc(c2l_feat, perm4.astype(jnp.int32), sd4, 20480,
                         320, 2)[:L_LITS]
    out = _readout(l_logit, ro_W1, ro_b1, ro_W2, ro_b2)
    return out + 0.0 * (num_edges + l_size)


# speedup vs baseline: 1.0764x; 1.0453x over previous
import functools

import jax
import jax.numpy as jnp
import numpy as np
from jax import lax
from jax.experimental import pallas as pl
from jax.experimental.pallas import tpu as pltpu
from jax.experimental.pallas import tpu_sc as plsc

DIM = 128
L_LITS = 20000

_EB = 128
_NW = 32


def _make_segop(mode, n_out, r_tile, n_pass, e_pad):
    rw = n_out // _NW
    assert rw == r_tile * n_pass and r_tile % 8 == 0
    nu = _NW * n_pass
    nu_pad = -(-(nu + 1) // 8) * 8
    mesh = plsc.VectorSubcoreMesh(core_axis_name="c", subcore_axis_name="s")
    if mode == "sum":
        out_type = jax.ShapeDtypeStruct((n_out, DIM), jnp.float32)
    else:
        out_type = (jax.ShapeDtypeStruct((n_out, DIM), jnp.float32),
                    jax.ShapeDtypeStruct((n_out, DIM), jnp.float32))
    scratch = [
        pltpu.VMEM((r_tile, DIM), jnp.float32),
        pltpu.VMEM_SHARED((384, DIM), jnp.float32),
        pltpu.VMEM((_EB, DIM), jnp.float32),
        pltpu.VMEM((_EB,), jnp.int32),
        pltpu.VMEM((_EB, 16), jnp.int32),
        pltpu.VMEM((nu_pad, 16), jnp.int32),
        pltpu.SemaphoreType.DMA,
    ]
    if mode == "lse":
        scratch.insert(2, pltpu.VMEM((r_tile, DIM), jnp.float32))
        scratch.insert(3, pltpu.VMEM_SHARED((384, DIM), jnp.float32))

    @functools.partial(pl.kernel, out_type=out_type, mesh=mesh,
                       compiler_params=pltpu.CompilerParams(
                           use_tc_tiling_on_sc=False),
                       scratch_types=scratch)
    def segop(feat, g, sd16, tb, *rest):
        if mode == "sum":
            (out, accum, zconst, rows_v, idx_v, sd_v, tb_v, sem) = rest
        else:
            (out_s, out_m, accum, zconst, amax, mconst, rows_v, idx_v, sd_v,
             tb_v, sem) = rest
        w = lax.axis_index("s") * 2 + lax.axis_index("c")
        pltpu.sync_copy(tb, tb_v)
        zv = jnp.zeros((16,), jnp.float32)

        def zbody(r, _):
            for j in range(DIM // 16):
                rows_v[r, pl.ds(j * 16, 16)] = zv
            return 0

        lax.fori_loop(0, _EB, zbody, 0)
        for off in (0, 128, 256):
            pltpu.sync_copy(rows_v, zconst.at[pl.ds(off, _EB)])
        if mode == "lse":
            mv = jnp.full((16,), -3e38, jnp.float32)

            def mbody(r, _):
                for j in range(DIM // 16):
                    rows_v[r, pl.ds(j * 16, 16)] = mv
                return 0

            lax.fori_loop(0, _EB, mbody, 0)
            for off in (0, 128, 256):
                pltpu.sync_copy(rows_v, mconst.at[pl.ds(off, _EB)])

        def unit_bounds(p):
            u = w * n_pass + p
            lo = tb_v[u, pl.ds(0, 16)][0]
            hi = tb_v[u + 1, pl.ds(0, 16)][0]
            return lo, hi

        def gather_block(bs):
            pltpu.sync_copy(g.at[pl.ds(bs, _EB)], idx_v)
            pltpu.sync_copy(sd16.at[pl.ds(bs, _EB)], sd_v)
            pltpu.async_copy(feat.at[idx_v], rows_v, sem).wait()

        def edge_range(bs, lo, hi):
            i0 = jnp.maximum(lo - bs, 0)
            i1 = jnp.minimum(hi - bs, _EB)
            return i0, i1

        def p_loop_sum(p, _):
            lo, hi = unit_bounds(p)
            base_row = w * rw + p * r_tile
            pltpu.sync_copy(zconst.at[pl.ds(0, r_tile)], accum)
            lo_al = (lo // _EB) * _EB
            nblk = (hi - lo_al + _EB - 1) // _EB

            def blk(bi, _c):
                bs = lo_al + bi * _EB
                gather_block(bs)
                i0, i1 = edge_range(bs, lo, hi)

                def edge(i, _e):
                    ldi = sd_v[i, pl.ds(0, 16)][0] - base_row
                    for j in range(DIM // 16):
                        sl = pl.ds(j * 16, 16)
                        accum[ldi, sl] = accum[ldi, sl] + rows_v[i, sl]
                    return 0

                lax.fori_loop(i0, i1, edge, 0)
                return 0

            lax.fori_loop(0, nblk, blk, 0)
            pltpu.sync_copy(
                accum,
                out.at[pl.ds(pl.multiple_of(base_row, 8), r_tile)])
            return 0

        def p_loop_lse(p, _):
            lo, hi = unit_bounds(p)
            base_row = w * rw + p * r_tile
            pltpu.sync_copy(zconst.at[pl.ds(0, r_tile)], accum)
            pltpu.sync_copy(mconst.at[pl.ds(0, r_tile)], amax)
            lo_al = (lo // _EB) * _EB
            nblk = (hi - lo_al + _EB - 1) // _EB

            def blkA(bi, _c):
                bs = lo_al + bi * _EB
                gather_block(bs)
                i0, i1 = edge_range(bs, lo, hi)

                def edge(i, _e):
                    ldi = sd_v[i, pl.ds(0, 16)][0] - base_row
                    for j in range(DIM // 16):
                        sl = pl.ds(j * 16, 16)
                        amax[ldi, sl] = jnp.maximum(amax[ldi, sl],
                                                    rows_v[i, sl])
                    return 0

                lax.fori_loop(i0, i1, edge, 0)
                return 0

            lax.fori_loop(0, nblk, blkA, 0)

            def blkB(bi, _c):
                bs = lo_al + bi * _EB
                gather_block(bs)
                i0, i1 = edge_range(bs, lo, hi)

                def edge(i, _e):
                    ldi = sd_v[i, pl.ds(0, 16)][0] - base_row
                    for j in range(DIM // 16):
                        sl = pl.ds(j * 16, 16)
                        accum[ldi, sl] = accum[ldi, sl] + jnp.exp(
                            rows_v[i, sl] - amax[ldi, sl])
                    return 0

                lax.fori_loop(i0, i1, edge, 0)
                return 0

            lax.fori_loop(0, nblk, blkB, 0)
            ob = pl.ds(pl.multiple_of(base_row, 8), r_tile)
            pltpu.sync_copy(accum, out_s.at[ob])
            pltpu.sync_copy(amax, out_m.at[ob])
            return 0

        lax.fori_loop(0, n_pass, p_loop_sum if mode == "sum" else p_loop_lse,
                      0)

    return segop


@functools.lru_cache(maxsize=None)
def _segop_fn(mode, n_out, r_tile, n_pass, e_pad):
    return jax.jit(_make_segop(mode, n_out, r_tile, n_pass, e_pad))


def _segop_sc(mode, feat, g, sd, n_out, r_tile, n_pass):
    e = g.shape[0]
    e_pad = e + _EB
    rw = n_out // _NW
    n_pass_i = n_pass
    nu = _NW * n_pass_i
    nu_pad = -(-(nu + 1) // 8) * 8
    u = jnp.arange(nu + 1, dtype=jnp.int32)
    row0 = (u // n_pass_i) * rw + (u % n_pass_i) * r_tile
    tb = jnp.searchsorted(sd, row0, side="left").astype(jnp.int32)
    tb = jnp.concatenate([tb, jnp.full((nu_pad - nu - 1,), e, jnp.int32)])
    tb = jnp.broadcast_to(tb[:, None], (nu_pad, 16))
    g_p = jnp.concatenate([g.astype(jnp.int32),
                           jnp.zeros((e_pad - e,), jnp.int32)])
    sd16 = jnp.broadcast_to(
        jnp.concatenate([sd.astype(jnp.int32),
                         jnp.zeros((e_pad - e,), jnp.int32)])[:, None],
        (e_pad, 16))
    fn = _segop_fn(mode, n_out, r_tile, n_pass_i, e_pad)
    return fn(feat, g_p, sd16, tb)


def _segsum_sc(feat, g, sd, n_out, r_tile, n_pass):
    return _segop_sc("sum", feat, g, sd, n_out, r_tile, n_pass)


def _seglse_sc(feat, g, sd, n_out, r_tile, n_pass):
    return _segop_sc("lse", feat, g, sd, n_out, r_tile, n_pass)


_BM = 640


def _pairswap(y, bm):
    down = pltpu.roll(y, bm - 1, 0)
    up = pltpu.roll(y, 1, 0)
    row = jax.lax.broadcasted_iota(jnp.int32, (bm, DIM), 0)
    return jnp.where((row % 2) == 0, down, up)


def _m1_body(x_ref, w1_ref, b1_ref, w2_ref, b2_ref, nw1_ref, nb1_ref,
             nw2_ref, nb2_ref, o_ref):
    x = x_ref[...]
    h = jnp.maximum(
        jnp.dot(x, w1_ref[...], preferred_element_type=jnp.float32)
        + b1_ref[...], 0.0)
    y = (jnp.dot(h, w2_ref[...], preferred_element_type=jnp.float32)
         + b2_ref[...])
    inv = _pairswap(y, x.shape[0])
    z = jnp.concatenate([y, inv], axis=1)
    h2 = jnp.maximum(
        jnp.dot(z, nw1_ref[...], preferred_element_type=jnp.float32)
        + nb1_ref[...], 0.0)
    o_ref[...] = (jnp.dot(h2, nw2_ref[...], preferred_element_type=jnp.float32)
                  + nb2_ref[...])


def _mlp_fused_l2c(x, w1, b1, w2, b2, nw1, nb1, nw2, nb2):
    n = x.shape[0]
    grid = n // _BM
    full = lambda s: pl.BlockSpec(s, lambda i: (0, 0))
    return pl.pallas_call(
        _m1_body,
        grid=(grid,),
        in_specs=[
            pl.BlockSpec((_BM, DIM), lambda i: (i, 0)),
            full((DIM, DIM)), full((1, DIM)), full((DIM, DIM)), full((1, DIM)),
            full((2 * DIM, DIM)), full((1, DIM)), full((DIM, DIM)), full((1, DIM)),
        ],
        out_specs=pl.BlockSpec((_BM, DIM), lambda i: (i, 0)),
        out_shape=jax.ShapeDtypeStruct((n, DIM), jnp.float32),
    )(x, w1, b1.reshape(1, DIM), w2, b2.reshape(1, DIM),
      nw1, nb1.reshape(1, DIM), nw2, nb2.reshape(1, DIM))


def _m2_body(s_ref, m_ref, w1_ref, b1_ref, w2_ref, b2_ref, o_ref):
    s = s_ref[...]
    m = m_ref[...]
    x = jnp.log(s + 1e-12) + jnp.where(s > 0, m, 0.0)
    h = jnp.maximum(
        jnp.dot(x, w1_ref[...], preferred_element_type=jnp.float32)
        + b1_ref[...], 0.0)
    o_ref[...] = (jnp.dot(h, w2_ref[...], preferred_element_type=jnp.float32)
                  + b2_ref[...])


def _mlp_fused_c2l(s, m, w1, b1, w2, b2):
    n = s.shape[0]
    grid = n // _BM
    full = lambda sh: pl.BlockSpec(sh, lambda i: (0, 0))
    return pl.pallas_call(
        _m2_body,
        grid=(grid,),
        in_specs=[
            pl.BlockSpec((_BM, DIM), lambda i: (i, 0)),
            pl.BlockSpec((_BM, DIM), lambda i: (i, 0)),
            full((DIM, DIM)), full((1, DIM)), full((DIM, DIM)), full((1, DIM)),
        ],
        out_specs=pl.BlockSpec((_BM, DIM), lambda i: (i, 0)),
        out_shape=jax.ShapeDtypeStruct((n, DIM), jnp.float32),
    )(s, m, w1, b1.reshape(1, DIM), w2, b2.reshape(1, DIM))


def _m3_body(x_ref, w1_ref, b1_ref, w2_ref, b2_ref, o_ref):
    x = x_ref[...]
    h = jnp.maximum(
        jnp.dot(x, w1_ref[...], preferred_element_type=jnp.float32)
        + b1_ref[...], 0.0)
    y = (jnp.dot(h, w2_ref[...], preferred_element_type=jnp.float32)
         + b2_ref[...])
    d = y - _pairswap(y, x.shape[0])
    o_ref[...] = jax.nn.sigmoid(d)


def _readout(x, w1, b1, w2, b2):
    w2p = jnp.pad(w2, ((0, 0), (0, DIM - 1)))
    b2p = jnp.pad(b2.reshape(1, 1), ((0, 0), (0, DIM - 1)))
    n = x.shape[0]
    bm = 400
    grid = n // bm
    full = lambda sh: pl.BlockSpec(sh, lambda i: (0, 0))
    out = pl.pallas_call(
        _m3_body,
        grid=(grid,),
        in_specs=[
            pl.BlockSpec((bm, DIM), lambda i: (i, 0)),
            full((DIM, DIM)), full((1, DIM)), full((DIM, DIM)), full((1, DIM)),
        ],
        out_specs=pl.BlockSpec((bm, DIM), lambda i: (i, 0)),
        out_shape=jax.ShapeDtypeStruct((n, DIM), jnp.float32),
    )(x, w1, b1.reshape(1, DIM), w2p, b2p)
    return out[:, 0].reshape(-1, 2)


def _segsum(feat, g, sd, n_out):
    return jax.ops.segment_sum(feat[g], sd, num_segments=n_out,
                               indices_are_sorted=True)


def _seg_logsumexp_parts(src_sorted, sd, n_out):
    seg_max = jax.ops.segment_max(src_sorted, sd, num_segments=n_out,
                                  indices_are_sorted=True)
    seg_max = jnp.where(jnp.isfinite(seg_max), seg_max, 0.0)
    seg_sum = jax.ops.segment_sum(jnp.exp(src_sorted - seg_max[sd]), sd,
                                  num_segments=n_out, indices_are_sorted=True)
    return seg_sum, seg_max


def kernel(sign_l_edge_index, c2l_msg_repeat_index, c2l_msg_scatter_index,
           l2c_msg_aggr_repeat_index, l2c_msg_aggr_scatter_index,
           l2c_msg_scatter_index, num_edges, l_size,
           c2l_init, l2c_init,
           c2l_W1, c2l_b1, c2l_W2, c2l_b2,
           l2c_W1, l2c_b1, l2c_W2, l2c_b2,
           nm_W1, nm_b1, nm_W2, nm_b2,
           ro_W1, ro_b1, ro_W2, ro_b2):
    E = sign_l_edge_index.shape[0]
    denom = np.sqrt(DIM)

    perm1 = jnp.argsort(c2l_msg_scatter_index)
    sd1 = c2l_msg_scatter_index[perm1]
    g1 = c2l_msg_repeat_index[perm1]

    perm2 = jnp.argsort(l2c_msg_aggr_scatter_index)
    sd2 = l2c_msg_aggr_scatter_index[perm2]
    g2 = l2c_msg_aggr_repeat_index[perm2]

    perm3 = jnp.argsort(l2c_msg_scatter_index)
    sd3 = l2c_msg_scatter_index[perm3]

    perm4 = jnp.argsort(sign_l_edge_index)
    sd4 = sign_l_edge_index[perm4]

    c2l_feat = jnp.tile(c2l_init / denom, (E, 1))

    for _ in range(2):
        c2l_msg = _segsum_sc(c2l_feat, g1, sd1, E, 200, 25)
        l2c_feat = _mlp_fused_l2c(c2l_msg, l2c_W1, l2c_b1, l2c_W2, l2c_b2,
                                  nm_W1, nm_b1, nm_W2, nm_b2)
        l2c_aggr = _segsum_sc(l2c_feat, g2, sd2, E, 200, 25)
        s, m = _seglse_sc(l2c_aggr, perm3.astype(jnp.int32), sd3, E, 200, 25)
        c2l_feat = _mlp_fused_c2l(s, m, c2l_W1, c2l_b1, c2l_W2, c2l_b2)

    l_logit = _segsum_sc(c2l_feat, perm4.astype(jnp.int32), sd4, 20480,
                         320, 2)[:L_LITS]
    out = _readout(l_logit, ro_W1, ro_b1, ro_W2, ro_b2)
    return out + 0.0 * (num_edges + l_size)


# speedup vs baseline: 1.1538x; 1.0719x over previous
import functools

import jax
import jax.numpy as jnp
import numpy as np
from jax import lax
from jax.experimental import pallas as pl
from jax.experimental.pallas import tpu as pltpu
from jax.experimental.pallas import tpu_sc as plsc

DIM = 128
L_LITS = 20000

_EB = 128
_NW = 32


def _make_segop(mode, n_out, r_tile, n_pass, e_pad):
    rw = n_out // _NW
    assert rw == r_tile * n_pass and r_tile % 8 == 0
    nu = _NW * n_pass
    nu_pad = -(-(nu + 1) // 8) * 8
    mesh = plsc.VectorSubcoreMesh(core_axis_name="c", subcore_axis_name="s")
    if mode == "sum":
        out_type = jax.ShapeDtypeStruct((n_out, DIM), jnp.float32)
    else:
        out_type = (jax.ShapeDtypeStruct((n_out, DIM), jnp.float32),
                    jax.ShapeDtypeStruct((n_out, DIM), jnp.float32))
    scratch = [
        pltpu.VMEM((r_tile, DIM), jnp.float32),
        pltpu.VMEM_SHARED((384, DIM), jnp.float32),
        pltpu.VMEM((_EB, DIM), jnp.float32),
        pltpu.VMEM((_EB,), jnp.int32),
        pltpu.VMEM((_EB, 16), jnp.int32),
        pltpu.VMEM((_EB, DIM), jnp.float32),
        pltpu.VMEM((_EB,), jnp.int32),
        pltpu.VMEM((_EB, 16), jnp.int32),
        pltpu.VMEM((nu_pad, 16), jnp.int32),
        pltpu.SemaphoreType.DMA,
        pltpu.SemaphoreType.DMA,
    ]
    if mode == "lse":
        scratch.insert(2, pltpu.VMEM((r_tile, DIM), jnp.float32))
        scratch.insert(3, pltpu.VMEM_SHARED((384, DIM), jnp.float32))

    @functools.partial(pl.kernel, out_type=out_type, mesh=mesh,
                       compiler_params=pltpu.CompilerParams(
                           use_tc_tiling_on_sc=False),
                       scratch_types=scratch)
    def segop(feat, g, sd16, tb, *rest):
        if mode == "sum":
            (out, accum, zconst, rows_v, idx_v, sd_v, rows_w, idx_w, sd_w,
             tb_v, sem, sem2) = rest
        else:
            (out_s, out_m, accum, zconst, amax, mconst, rows_v, idx_v, sd_v,
             rows_w, idx_w, sd_w, tb_v, sem, sem2) = rest
        bufs = ((rows_v, idx_v, sd_v, sem), (rows_w, idx_w, sd_w, sem2))
        w = lax.axis_index("s") * 2 + lax.axis_index("c")
        pltpu.sync_copy(tb, tb_v)
        zv = jnp.zeros((16,), jnp.float32)

        def zbody(r, _):
            for j in range(DIM // 16):
                rows_v[r, pl.ds(j * 16, 16)] = zv
            return 0

        lax.fori_loop(0, _EB, zbody, 0)
        for off in (0, 128, 256):
            pltpu.sync_copy(rows_v, zconst.at[pl.ds(off, _EB)])
        if mode == "lse":
            mv = jnp.full((16,), -3e38, jnp.float32)

            def mbody(r, _):
                for j in range(DIM // 16):
                    rows_v[r, pl.ds(j * 16, 16)] = mv
                return 0

            lax.fori_loop(0, _EB, mbody, 0)
            for off in (0, 128, 256):
                pltpu.sync_copy(rows_v, mconst.at[pl.ds(off, _EB)])

        def unit_bounds(p):
            u = w * n_pass + p
            lo = tb_v[u, pl.ds(0, 16)][0]
            hi = tb_v[u + 1, pl.ds(0, 16)][0]
            return lo, hi

        def start_gather(bs, b):
            rows_b, idx_b, sd_b, sem_b = bufs[b]
            pltpu.sync_copy(g.at[pl.ds(bs, _EB)], idx_b)
            pltpu.sync_copy(sd16.at[pl.ds(bs, _EB)], sd_b)
            pltpu.async_copy(feat.at[idx_b], rows_b, sem_b)

        def wait_gather(b):
            rows_b, idx_b, sd_b, sem_b = bufs[b]
            pltpu.make_async_copy(feat.at[idx_b], rows_b, sem_b).wait()

        def edge_range(bs, lo, hi):
            i0 = jnp.maximum(lo - bs, 0)
            i1 = jnp.minimum(hi - bs, _EB)
            return i0, i1

        def run_blocks(lo_al, nblk, lo, hi, process):
            @pl.when(nblk > 0)
            def _():
                start_gather(lo_al, 0)

            def pair(b2, _):
                bi0 = 2 * b2
                bs0 = lo_al + bi0 * _EB
                wait_gather(0)

                @pl.when(bi0 + 1 < nblk)
                def _():
                    start_gather(bs0 + _EB, 1)

                i0, i1 = edge_range(bs0, lo, hi)
                process(0, i0, i1)

                @pl.when(bi0 + 1 < nblk)
                def _():
                    wait_gather(1)

                    @pl.when(bi0 + 2 < nblk)
                    def _():
                        start_gather(bs0 + 2 * _EB, 0)

                    j0, j1 = edge_range(bs0 + _EB, lo, hi)
                    process(1, j0, j1)

                return 0

            lax.fori_loop(0, (nblk + 1) // 2, pair, 0)

        def p_loop_sum(p, _):
            lo, hi = unit_bounds(p)
            base_row = w * rw + p * r_tile
            pltpu.sync_copy(zconst.at[pl.ds(0, r_tile)], accum)
            lo_al = (lo // _EB) * _EB
            nblk = (hi - lo_al + _EB - 1) // _EB

            def process(b, i0, i1):
                rows_b, _, sd_b, _s = bufs[b]

                def edge(i, _e):
                    ldi = sd_b[i, pl.ds(0, 16)][0] - base_row
                    for j in range(DIM // 16):
                        sl = pl.ds(j * 16, 16)
                        accum[ldi, sl] = accum[ldi, sl] + rows_b[i, sl]
                    return 0

                lax.fori_loop(i0, i1, edge, 0)

            run_blocks(lo_al, nblk, lo, hi, process)
            pltpu.sync_copy(
                accum,
                out.at[pl.ds(pl.multiple_of(base_row, 8), r_tile)])
            return 0

        def p_loop_lse(p, _):
            lo, hi = unit_bounds(p)
            base_row = w * rw + p * r_tile
            pltpu.sync_copy(zconst.at[pl.ds(0, r_tile)], accum)
            pltpu.sync_copy(mconst.at[pl.ds(0, r_tile)], amax)
            lo_al = (lo // _EB) * _EB
            nblk = (hi - lo_al + _EB - 1) // _EB

            def processA(b, i0, i1):
                rows_b, _, sd_b, _s = bufs[b]

                def edge(i, _e):
                    ldi = sd_b[i, pl.ds(0, 16)][0] - base_row
                    for j in range(DIM // 16):
                        sl = pl.ds(j * 16, 16)
                        amax[ldi, sl] = jnp.maximum(amax[ldi, sl],
                                                    rows_b[i, sl])
                    return 0

                lax.fori_loop(i0, i1, edge, 0)

            run_blocks(lo_al, nblk, lo, hi, processA)

            def processB(b, i0, i1):
                rows_b, _, sd_b, _s = bufs[b]

                def edge(i, _e):
                    ldi = sd_b[i, pl.ds(0, 16)][0] - base_row
                    for j in range(DIM // 16):
                        sl = pl.ds(j * 16, 16)
                        accum[ldi, sl] = accum[ldi, sl] + jnp.exp(
                            rows_b[i, sl] - amax[ldi, sl])
                    return 0

                lax.fori_loop(i0, i1, edge, 0)

            run_blocks(lo_al, nblk, lo, hi, processB)
            ob = pl.ds(pl.multiple_of(base_row, 8), r_tile)
            pltpu.sync_copy(accum, out_s.at[ob])
            pltpu.sync_copy(amax, out_m.at[ob])
            return 0

        lax.fori_loop(0, n_pass, p_loop_sum if mode == "sum" else p_loop_lse,
                      0)

    return segop


@functools.lru_cache(maxsize=None)
def _segop_fn(mode, n_out, r_tile, n_pass, e_pad):
    return jax.jit(_make_segop(mode, n_out, r_tile, n_pass, e_pad))


def _segop_sc(mode, feat, g, sd, n_out, r_tile, n_pass):
    e = g.shape[0]
    e_pad = e + _EB
    rw = n_out // _NW
    n_pass_i = n_pass
    nu = _NW * n_pass_i
    nu_pad = -(-(nu + 1) // 8) * 8
    u = jnp.arange(nu + 1, dtype=jnp.int32)
    row0 = (u // n_pass_i) * rw + (u % n_pass_i) * r_tile
    tb = jnp.searchsorted(sd, row0, side="left").astype(jnp.int32)
    tb = jnp.concatenate([tb, jnp.full((nu_pad - nu - 1,), e, jnp.int32)])
    tb = jnp.broadcast_to(tb[:, None], (nu_pad, 16))
    g_p = jnp.concatenate([g.astype(jnp.int32),
                           jnp.zeros((e_pad - e,), jnp.int32)])
    sd16 = jnp.broadcast_to(
        jnp.concatenate([sd.astype(jnp.int32),
                         jnp.zeros((e_pad - e,), jnp.int32)])[:, None],
        (e_pad, 16))
    fn = _segop_fn(mode, n_out, r_tile, n_pass_i, e_pad)
    return fn(feat, g_p, sd16, tb)


def _segsum_sc(feat, g, sd, n_out, r_tile, n_pass):
    return _segop_sc("sum", feat, g, sd, n_out, r_tile, n_pass)


def _seglse_sc(feat, g, sd, n_out, r_tile, n_pass):
    return _segop_sc("lse", feat, g, sd, n_out, r_tile, n_pass)


_BM = 640


def _pairswap(y, bm):
    down = pltpu.roll(y, bm - 1, 0)
    up = pltpu.roll(y, 1, 0)
    row = jax.lax.broadcasted_iota(jnp.int32, (bm, DIM), 0)
    return jnp.where((row % 2) == 0, down, up)


def _m1_body(x_ref, w1_ref, b1_ref, w2_ref, b2_ref, nw1_ref, nb1_ref,
             nw2_ref, nb2_ref, o_ref):
    x = x_ref[...]
    h = jnp.maximum(
        jnp.dot(x, w1_ref[...], preferred_element_type=jnp.float32)
        + b1_ref[...], 0.0)
    y = (jnp.dot(h, w2_ref[...], preferred_element_type=jnp.float32)
         + b2_ref[...])
    inv = _pairswap(y, x.shape[0])
    z = jnp.concatenate([y, inv], axis=1)
    h2 = jnp.maximum(
        jnp.dot(z, nw1_ref[...], preferred_element_type=jnp.float32)
        + nb1_ref[...], 0.0)
    o_ref[...] = (jnp.dot(h2, nw2_ref[...], preferred_element_type=jnp.float32)
                  + nb2_ref[...])


def _mlp_fused_l2c(x, w1, b1, w2, b2, nw1, nb1, nw2, nb2):
    n = x.shape[0]
    grid = n // _BM
    full = lambda s: pl.BlockSpec(s, lambda i: (0, 0))
    return pl.pallas_call(
        _m1_body,
        grid=(grid,),
        in_specs=[
            pl.BlockSpec((_BM, DIM), lambda i: (i, 0)),
            full((DIM, DIM)), full((1, DIM)), full((DIM, DIM)), full((1, DIM)),
            full((2 * DIM, DIM)), full((1, DIM)), full((DIM, DIM)), full((1, DIM)),
        ],
        out_specs=pl.BlockSpec((_BM, DIM), lambda i: (i, 0)),
        out_shape=jax.ShapeDtypeStruct((n, DIM), jnp.float32),
    )(x, w1, b1.reshape(1, DIM), w2, b2.reshape(1, DIM),
      nw1, nb1.reshape(1, DIM), nw2, nb2.reshape(1, DIM))


def _m2_body(s_ref, m_ref, w1_ref, b1_ref, w2_ref, b2_ref, o_ref):
    s = s_ref[...]
    m = m_ref[...]
    x = jnp.log(s + 1e-12) + jnp.where(s > 0, m, 0.0)
    h = jnp.maximum(
        jnp.dot(x, w1_ref[...], preferred_element_type=jnp.float32)
        + b1_ref[...], 0.0)
    o_ref[...] = (jnp.dot(h, w2_ref[...], preferred_element_type=jnp.float32)
                  + b2_ref[...])


def _mlp_fused_c2l(s, m, w1, b1, w2, b2):
    n = s.shape[0]
    grid = n // _BM
    full = lambda sh: pl.BlockSpec(sh, lambda i: (0, 0))
    return pl.pallas_call(
        _m2_body,
        grid=(grid,),
        in_specs=[
            pl.BlockSpec((_BM, DIM), lambda i: (i, 0)),
            pl.BlockSpec((_BM, DIM), lambda i: (i, 0)),
            full((DIM, DIM)), full((1, DIM)), full((DIM, DIM)), full((1, DIM)),
        ],
        out_specs=pl.BlockSpec((_BM, DIM), lambda i: (i, 0)),
        out_shape=jax.ShapeDtypeStruct((n, DIM), jnp.float32),
    )(s, m, w1, b1.reshape(1, DIM), w2, b2.reshape(1, DIM))


def _m3_body(x_ref, w1_ref, b1_ref, w2_ref, b2_ref, o_ref):
    x = x_ref[...]
    h = jnp.maximum(
        jnp.dot(x, w1_ref[...], preferred_element_type=jnp.float32)
        + b1_ref[...], 0.0)
    y = (jnp.dot(h, w2_ref[...], preferred_element_type=jnp.float32)
         + b2_ref[...])
    d = y - _pairswap(y, x.shape[0])
    o_ref[...] = jax.nn.sigmoid(d)


def _readout(x, w1, b1, w2, b2):
    w2p = jnp.pad(w2, ((0, 0), (0, DIM - 1)))
    b2p = jnp.pad(b2.reshape(1, 1), ((0, 0), (0, DIM - 1)))
    n = x.shape[0]
    bm = 400
    grid = n // bm
    full = lambda sh: pl.BlockSpec(sh, lambda i: (0, 0))
    out = pl.pallas_call(
        _m3_body,
        grid=(grid,),
        in_specs=[
            pl.BlockSpec((bm, DIM), lambda i: (i, 0)),
            full((DIM, DIM)), full((1, DIM)), full((DIM, DIM)), full((1, DIM)),
        ],
        out_specs=pl.BlockSpec((bm, DIM), lambda i: (i, 0)),
        out_shape=jax.ShapeDtypeStruct((n, DIM), jnp.float32),
    )(x, w1, b1.reshape(1, DIM), w2p, b2p)
    return out[:, 0].reshape(-1, 2)


def _segsum(feat, g, sd, n_out):
    return jax.ops.segment_sum(feat[g], sd, num_segments=n_out,
                               indices_are_sorted=True)


def _seg_logsumexp_parts(src_sorted, sd, n_out):
    seg_max = jax.ops.segment_max(src_sorted, sd, num_segments=n_out,
                                  indices_are_sorted=True)
    seg_max = jnp.where(jnp.isfinite(seg_max), seg_max, 0.0)
    seg_sum = jax.ops.segment_sum(jnp.exp(src_sorted - seg_max[sd]), sd,
                                  num_segments=n_out, indices_are_sorted=True)
    return seg_sum, seg_max


def kernel(sign_l_edge_index, c2l_msg_repeat_index, c2l_msg_scatter_index,
           l2c_msg_aggr_repeat_index, l2c_msg_aggr_scatter_index,
           l2c_msg_scatter_index, num_edges, l_size,
           c2l_init, l2c_init,
           c2l_W1, c2l_b1, c2l_W2, c2l_b2,
           l2c_W1, l2c_b1, l2c_W2, l2c_b2,
           nm_W1, nm_b1, nm_W2, nm_b2,
           ro_W1, ro_b1, ro_W2, ro_b2):
    E = sign_l_edge_index.shape[0]
    denom = np.sqrt(DIM)

    perm1 = jnp.argsort(c2l_msg_scatter_index)
    sd1 = c2l_msg_scatter_index[perm1]
    g1 = c2l_msg_repeat_index[perm1]

    perm2 = jnp.argsort(l2c_msg_aggr_scatter_index)
    sd2 = l2c_msg_aggr_scatter_index[perm2]
    g2 = l2c_msg_aggr_repeat_index[perm2]

    perm3 = jnp.argsort(l2c_msg_scatter_index)
    sd3 = l2c_msg_scatter_index[perm3]

    perm4 = jnp.argsort(sign_l_edge_index)
    sd4 = sign_l_edge_index[perm4]

    c2l_feat = jnp.tile(c2l_init / denom, (E, 1))

    for _ in range(2):
        c2l_msg = _segsum_sc(c2l_feat, g1, sd1, E, 200, 25)
        l2c_feat = _mlp_fused_l2c(c2l_msg, l2c_W1, l2c_b1, l2c_W2, l2c_b2,
                                  nm_W1, nm_b1, nm_W2, nm_b2)
        l2c_aggr = _segsum_sc(l2c_feat, g2, sd2, E, 200, 25)
        s, m = _seglse_sc(l2c_aggr, perm3.astype(jnp.int32), sd3, E, 200, 25)
        c2l_feat = _mlp_fused_c2l(s, m, c2l_W1, c2l_b1, c2l_W2, c2l_b2)

    l_logit = _segsum_sc(c2l_feat, perm4.astype(jnp.int32), sd4, 20480,
                         320, 2)[:L_LITS]
    out = _readout(l_logit, ro_W1, ro_b1, ro_W2, ro_b2)
    return out + 0.0 * (num_edges + l_size)


# speedup vs baseline: 1.1539x; 1.0000x over previous
import functools

import jax
import jax.numpy as jnp
import numpy as np
from jax import lax
from jax.experimental import pallas as pl
from jax.experimental.pallas import tpu as pltpu
from jax.experimental.pallas import tpu_sc as plsc

DIM = 128
L_LITS = 20000

_EB = 128
_NW = 32


def _make_segop(mode, n_out, r_tile, n_pass, e_pad):
    rw = n_out // _NW
    assert rw == r_tile * n_pass and r_tile % 8 == 0
    nu = _NW * n_pass
    nu_pad = -(-(nu + 1) // 8) * 8
    mesh = plsc.VectorSubcoreMesh(core_axis_name="c", subcore_axis_name="s")
    if mode == "sum":
        out_type = jax.ShapeDtypeStruct((n_out, DIM), jnp.float32)
    else:
        out_type = (jax.ShapeDtypeStruct((n_out, DIM), jnp.float32),
                    jax.ShapeDtypeStruct((n_out, DIM), jnp.float32))
    scratch = [
        pltpu.VMEM((r_tile, DIM), jnp.float32),
        pltpu.VMEM_SHARED((384, DIM), jnp.float32),
        pltpu.VMEM((_EB, DIM), jnp.float32),
        pltpu.VMEM((_EB,), jnp.int32),
        pltpu.VMEM((_EB, 16), jnp.int32),
        pltpu.VMEM((_EB, DIM), jnp.float32),
        pltpu.VMEM((_EB,), jnp.int32),
        pltpu.VMEM((_EB, 16), jnp.int32),
        pltpu.VMEM((nu_pad, 16), jnp.int32),
        pltpu.SemaphoreType.DMA,
        pltpu.SemaphoreType.DMA,
    ]
    if mode == "lse":
        scratch.insert(2, pltpu.VMEM((r_tile, DIM), jnp.float32))
        scratch.insert(3, pltpu.VMEM_SHARED((384, DIM), jnp.float32))

    @functools.partial(pl.kernel, out_type=out_type, mesh=mesh,
                       compiler_params=pltpu.CompilerParams(
                           use_tc_tiling_on_sc=False),
                       scratch_types=scratch)
    def segop(feat, g, sd16, tb, *rest):
        if mode == "sum":
            (out, accum, zconst, rows_v, idx_v, sd_v, rows_w, idx_w, sd_w,
             tb_v, sem, sem2) = rest
        else:
            (out_s, out_m, accum, zconst, amax, mconst, rows_v, idx_v, sd_v,
             rows_w, idx_w, sd_w, tb_v, sem, sem2) = rest
        bufs = ((rows_v, idx_v, sd_v, sem), (rows_w, idx_w, sd_w, sem2))
        w = lax.axis_index("s") * 2 + lax.axis_index("c")
        pltpu.sync_copy(tb, tb_v)
        zv = jnp.zeros((16,), jnp.float32)

        def zbody(r, _):
            for j in range(DIM // 16):
                rows_v[r, pl.ds(j * 16, 16)] = zv
            return 0

        lax.fori_loop(0, _EB, zbody, 0)
        for off in (0, 128, 256):
            pltpu.sync_copy(rows_v, zconst.at[pl.ds(off, _EB)])
        if mode == "lse":
            mv = jnp.full((16,), -3e38, jnp.float32)

            def mbody(r, _):
                for j in range(DIM // 16):
                    rows_v[r, pl.ds(j * 16, 16)] = mv
                return 0

            lax.fori_loop(0, _EB, mbody, 0)
            for off in (0, 128, 256):
                pltpu.sync_copy(rows_v, mconst.at[pl.ds(off, _EB)])

        def unit_bounds(p):
            u = w * n_pass + p
            lo = tb_v[u, pl.ds(0, 16)][0]
            hi = tb_v[u + 1, pl.ds(0, 16)][0]
            return lo, hi

        def start_gather(bs, b):
            rows_b, idx_b, sd_b, sem_b = bufs[b]
            pltpu.sync_copy(g.at[pl.ds(bs, _EB)], idx_b)
            pltpu.sync_copy(sd16.at[pl.ds(bs, _EB)], sd_b)
            pltpu.async_copy(feat.at[idx_b], rows_b, sem_b)

        def wait_gather(b):
            rows_b, idx_b, sd_b, sem_b = bufs[b]
            pltpu.make_async_copy(feat.at[idx_b], rows_b, sem_b).wait()

        def edge_range(bs, lo, hi):
            i0 = jnp.maximum(lo - bs, 0)
            i1 = jnp.minimum(hi - bs, _EB)
            return i0, i1

        def run_blocks(lo_al, nblk, lo, hi, process):
            @pl.when(nblk > 0)
            def _():
                start_gather(lo_al, 0)

            def pair(b2, _):
                bi0 = 2 * b2
                bs0 = lo_al + bi0 * _EB
                wait_gather(0)

                @pl.when(bi0 + 1 < nblk)
                def _():
                    start_gather(bs0 + _EB, 1)

                i0, i1 = edge_range(bs0, lo, hi)
                process(0, i0, i1)

                @pl.when(bi0 + 1 < nblk)
                def _():
                    wait_gather(1)

                    @pl.when(bi0 + 2 < nblk)
                    def _():
                        start_gather(bs0 + 2 * _EB, 0)

                    j0, j1 = edge_range(bs0 + _EB, lo, hi)
                    process(1, j0, j1)

                return 0

            lax.fori_loop(0, (nblk + 1) // 2, pair, 0)

        def p_loop_sum(p, _):
            lo, hi = unit_bounds(p)
            base_row = w * rw + p * r_tile
            pltpu.sync_copy(zconst.at[pl.ds(0, r_tile)], accum)
            lo_al = (lo // _EB) * _EB
            nblk = (hi - lo_al + _EB - 1) // _EB

            def process(b, i0, i1):
                rows_b, _, sd_b, _s = bufs[b]

                def edge(i, _e):
                    ldi = sd_b[i, pl.ds(0, 16)][0] - base_row
                    for j in range(DIM // 16):
                        sl = pl.ds(j * 16, 16)
                        accum[ldi, sl] = accum[ldi, sl] + rows_b[i, sl]
                    return 0

                lax.fori_loop(i0, i1, edge, 0)

            run_blocks(lo_al, nblk, lo, hi, process)
            pltpu.sync_copy(
                accum,
                out.at[pl.ds(pl.multiple_of(base_row, 8), r_tile)])
            return 0

        def p_loop_lse(p, _):
            lo, hi = unit_bounds(p)
            base_row = w * rw + p * r_tile
            pltpu.sync_copy(zconst.at[pl.ds(0, r_tile)], accum)
            pltpu.sync_copy(mconst.at[pl.ds(0, r_tile)], amax)
            lo_al = (lo // _EB) * _EB
            nblk = (hi - lo_al + _EB - 1) // _EB

            def processA(b, i0, i1):
                rows_b, _, sd_b, _s = bufs[b]

                def edge(i, _e):
                    ldi = sd_b[i, pl.ds(0, 16)][0] - base_row
                    for j in range(DIM // 16):
                        sl = pl.ds(j * 16, 16)
                        amax[ldi, sl] = jnp.maximum(amax[ldi, sl],
                                                    rows_b[i, sl])
                    return 0

                lax.fori_loop(i0, i1, edge, 0)

            run_blocks(lo_al, nblk, lo, hi, processA)

            def processB(b, i0, i1):
                rows_b, _, sd_b, _s = bufs[b]

                def edge(i, _e):
                    ldi = sd_b[i, pl.ds(0, 16)][0] - base_row
                    for j in range(DIM // 16):
                        sl = pl.ds(j * 16, 16)
                        accum[ldi, sl] = accum[ldi, sl] + jnp.exp(
                            rows_b[i, sl] - amax[ldi, sl])
                    return 0

                lax.fori_loop(i0, i1, edge, 0)

            run_blocks(lo_al, nblk, lo, hi, processB)
            ob = pl.ds(pl.multiple_of(base_row, 8), r_tile)
            pltpu.sync_copy(accum, out_s.at[ob])
            pltpu.sync_copy(amax, out_m.at[ob])
            return 0

        lax.fori_loop(0, n_pass, p_loop_sum if mode == "sum" else p_loop_lse,
                      0)

    return segop


@functools.lru_cache(maxsize=None)
def _segop_fn(mode, n_out, r_tile, n_pass, e_pad):
    return jax.jit(_make_segop(mode, n_out, r_tile, n_pass, e_pad))


def _segop_sc(mode, feat, g, sd, n_out, r_tile, n_pass):
    e = g.shape[0]
    e_pad = e + _EB
    rw = n_out // _NW
    n_pass_i = n_pass
    nu = _NW * n_pass_i
    nu_pad = -(-(nu + 1) // 8) * 8
    u = jnp.arange(nu + 1, dtype=jnp.int32)
    row0 = (u // n_pass_i) * rw + (u % n_pass_i) * r_tile
    tb = jnp.searchsorted(sd, row0, side="left").astype(jnp.int32)
    tb = jnp.concatenate([tb, jnp.full((nu_pad - nu - 1,), e, jnp.int32)])
    tb = jnp.broadcast_to(tb[:, None], (nu_pad, 16))
    g_p = jnp.concatenate([g.astype(jnp.int32),
                           jnp.zeros((e_pad - e,), jnp.int32)])
    sd16 = jnp.broadcast_to(
        jnp.concatenate([sd.astype(jnp.int32),
                         jnp.zeros((e_pad - e,), jnp.int32)])[:, None],
        (e_pad, 16))
    fn = _segop_fn(mode, n_out, r_tile, n_pass_i, e_pad)
    return fn(feat, g_p, sd16, tb)


def _segsum_sc(feat, g, sd, n_out, r_tile, n_pass):
    return _segop_sc("sum", feat, g, sd, n_out, r_tile, n_pass)


def _seglse_sc(feat, g, sd, n_out, r_tile, n_pass):
    return _segop_sc("lse", feat, g, sd, n_out, r_tile, n_pass)


_BM = 640


def _pairswap(y, bm):
    down = pltpu.roll(y, bm - 1, 0)
    up = pltpu.roll(y, 1, 0)
    row = jax.lax.broadcasted_iota(jnp.int32, (bm, DIM), 0)
    return jnp.where((row % 2) == 0, down, up)


def _m1_body(x_ref, w1_ref, b1_ref, w2_ref, b2_ref, nw1_ref, nb1_ref,
             nw2_ref, nb2_ref, o_ref):
    x = x_ref[...]
    h = jnp.maximum(
        jnp.dot(x, w1_ref[...], preferred_element_type=jnp.float32)
        + b1_ref[...], 0.0)
    y = (jnp.dot(h, w2_ref[...], preferred_element_type=jnp.float32)
         + b2_ref[...])
    inv = _pairswap(y, x.shape[0])
    z = jnp.concatenate([y, inv], axis=1)
    h2 = jnp.maximum(
        jnp.dot(z, nw1_ref[...], preferred_element_type=jnp.float32)
        + nb1_ref[...], 0.0)
    o_ref[...] = (jnp.dot(h2, nw2_ref[...], preferred_element_type=jnp.float32)
                  + nb2_ref[...])


def _mlp_fused_l2c(x, w1, b1, w2, b2, nw1, nb1, nw2, nb2):
    n = x.shape[0]
    grid = n // _BM
    full = lambda s: pl.BlockSpec(s, lambda i: (0, 0))
    return pl.pallas_call(
        _m1_body,
        grid=(grid,),
        in_specs=[
            pl.BlockSpec((_BM, DIM), lambda i: (i, 0)),
            full((DIM, DIM)), full((1, DIM)), full((DIM, DIM)), full((1, DIM)),
            full((2 * DIM, DIM)), full((1, DIM)), full((DIM, DIM)), full((1, DIM)),
        ],
        out_specs=pl.BlockSpec((_BM, DIM), lambda i: (i, 0)),
        out_shape=jax.ShapeDtypeStruct((n, DIM), jnp.float32),
    )(x, w1, b1.reshape(1, DIM), w2, b2.reshape(1, DIM),
      nw1, nb1.reshape(1, DIM), nw2, nb2.reshape(1, DIM))


def _m2_body(s_ref, m_ref, w1_ref, b1_ref, w2_ref, b2_ref, o_ref):
    s = s_ref[...]
    m = m_ref[...]
    x = jnp.log(s + 1e-12) + jnp.where(s > 0, m, 0.0)
    h = jnp.maximum(
        jnp.dot(x, w1_ref[...], preferred_element_type=jnp.float32)
        + b1_ref[...], 0.0)
    o_ref[...] = (jnp.dot(h, w2_ref[...], preferred_element_type=jnp.float32)
                  + b2_ref[...])


def _mlp_fused_c2l(s, m, w1, b1, w2, b2):
    n = s.shape[0]
    grid = n // _BM
    full = lambda sh: pl.BlockSpec(sh, lambda i: (0, 0))
    return pl.pallas_call(
        _m2_body,
        grid=(grid,),
        in_specs=[
            pl.BlockSpec((_BM, DIM), lambda i: (i, 0)),
            pl.BlockSpec((_BM, DIM), lambda i: (i, 0)),
            full((DIM, DIM)), full((1, DIM)), full((DIM, DIM)), full((1, DIM)),
        ],
        out_specs=pl.BlockSpec((_BM, DIM), lambda i: (i, 0)),
        out_shape=jax.ShapeDtypeStruct((n, DIM), jnp.float32),
    )(s, m, w1, b1.reshape(1, DIM), w2, b2.reshape(1, DIM))


def _m3_body(x_ref, w1_ref, b1_ref, w2_ref, b2_ref, o_ref):
    x = x_ref[...]
    h = jnp.maximum(
        jnp.dot(x, w1_ref[...], preferred_element_type=jnp.float32)
        + b1_ref[...], 0.0)
    y = (jnp.dot(h, w2_ref[...], preferred_element_type=jnp.float32)
         + b2_ref[...])
    d = y - _pairswap(y, x.shape[0])
    o_ref[...] = jax.nn.sigmoid(d)


def _readout(x, w1, b1, w2, b2):
    w2p = jnp.pad(w2, ((0, 0), (0, DIM - 1)))
    b2p = jnp.pad(b2.reshape(1, 1), ((0, 0), (0, DIM - 1)))
    n = x.shape[0]
    bm = 400
    grid = n // bm
    full = lambda sh: pl.BlockSpec(sh, lambda i: (0, 0))
    out = pl.pallas_call(
        _m3_body,
        grid=(grid,),
        in_specs=[
            pl.BlockSpec((bm, DIM), lambda i: (i, 0)),
            full((DIM, DIM)), full((1, DIM)), full((DIM, DIM)), full((1, DIM)),
        ],
        out_specs=pl.BlockSpec((bm, DIM), lambda i: (i, 0)),
        out_shape=jax.ShapeDtypeStruct((n, DIM), jnp.float32),
    )(x, w1, b1.reshape(1, DIM), w2p, b2p)
    return out[:, 0].reshape(-1, 2)


def kernel(sign_l_edge_index, c2l_msg_repeat_index, c2l_msg_scatter_index,
           l2c_msg_aggr_repeat_index, l2c_msg_aggr_scatter_index,
           l2c_msg_scatter_index, num_edges, l_size,
           c2l_init, l2c_init,
           c2l_W1, c2l_b1, c2l_W2, c2l_b2,
           l2c_W1, l2c_b1, l2c_W2, l2c_b2,
           nm_W1, nm_b1, nm_W2, nm_b2,
           ro_W1, ro_b1, ro_W2, ro_b2):
    E = sign_l_edge_index.shape[0]
    denom = np.sqrt(DIM)

    perm1 = jnp.argsort(c2l_msg_scatter_index)
    sd1 = c2l_msg_scatter_index[perm1]
    g1 = c2l_msg_repeat_index[perm1]

    perm2 = jnp.argsort(l2c_msg_aggr_scatter_index)
    sd2 = l2c_msg_aggr_scatter_index[perm2]
    g2 = l2c_msg_aggr_repeat_index[perm2]

    perm3 = jnp.argsort(l2c_msg_scatter_index)
    sd3 = l2c_msg_scatter_index[perm3]

    perm4 = jnp.argsort(sign_l_edge_index)
    sd4 = sign_l_edge_index[perm4]

    c2l_feat = jnp.tile(c2l_init / denom, (E, 1))

    for _ in range(2):
        c2l_msg = _segsum_sc(c2l_feat, g1, sd1, E, 200, 25)
        l2c_feat = _mlp_fused_l2c(c2l_msg, l2c_W1, l2c_b1, l2c_W2, l2c_b2,
                                  nm_W1, nm_b1, nm_W2, nm_b2)
        l2c_aggr = _segsum_sc(l2c_feat, g2, sd2, E, 200, 25)
        s, m = _seglse_sc(l2c_aggr, perm3.astype(jnp.int32), sd3, E, 200, 25)
        c2l_feat = _mlp_fused_c2l(s, m, c2l_W1, c2l_b1, c2l_W2, c2l_b2)

    l_logit = _segsum_sc(c2l_feat, perm4.astype(jnp.int32), sd4, 20480,
                         320, 2)[:L_LITS]
    out = _readout(l_logit, ro_W1, ro_b1, ro_W2, ro_b2)
    return out + 0.0 * (num_edges + l_size)
